# baseline (speedup 1.0000x reference)
import jax
import jax.numpy as jnp
import numpy as np
from jax import lax
from jax.experimental import pallas as pl
from jax.experimental.pallas import tpu as pltpu
from jax.experimental.pallas import tpu_sc as plsc

B = 4096
D_NUM = 100
D_PAD = 112
N_CAT = 26
CAT_SIZE = 1000
D_TOKEN = 128
N_TOK = D_NUM + N_CAT
SLAB = 132

NC = 2
NS = 16
NW = NC * NS
BW = B // NW
L = 16
RING = 4


def _sc_body(xcat_hbm, xT_hbm, w_hbm, table_hbm, out_hbm,
             w_v, x_v, idx_v, s0, s1, s2, s3,
             sem_a, sem_w0, sem_w1, sem_w2, sem_w3):
    w = lax.axis_index("c") * NS + lax.axis_index("s")

    pltpu.sync_copy(w_hbm, w_v)
    pltpu.sync_copy(xT_hbm.at[w], x_v)
    pltpu.sync_copy(xcat_hbm.at[w], idx_v)

    iota = lax.iota(jnp.int32, L)
    offA = iota * CAT_SIZE
    offB = jnp.where(iota < N_CAT - L, (iota + L) * CAT_SIZE, 0)

    def add_off(r, carry):
        sA = pl.ds(0, L)
        sB = pl.ds(L, L)
        idx_v[r, sA] = idx_v[r, sA] + offA
        idx_v[r, sB] = idx_v[r, sB] + offB
        return carry

    lax.fori_loop(0, BW, add_off, 0)

    slabs = (s0, s1, s2, s3)
    sem_w = (sem_w0, sem_w1, sem_w2, sem_w3)
    lane_ids = [jnp.full((L,), i, dtype=jnp.int32) for i in range(L)]

    def emit_rows(slab, bl, k, n_dd, x_off=None, lane_off=0):
        xk = x_v[bl, pl.ds(k * L if x_off is None else x_off, L)]
        for dd in range(n_dd):
            d = k * L + dd
            splat = jnp.take(xk, lane_ids[lane_off + dd])
            for g in range(D_TOKEN // L):
                s = pl.ds(g * L, L)
                slab[d, s] = splat * w_v[d, s]

    def do_slab(p, t):
        bl = p * RING + t
        slab = slabs[t]
        b = w * BW + bl

        @pl.when(p > 0)
        def _drain():
            pltpu.make_async_copy(
                slab.at[pl.ds(0, N_TOK)], out_hbm.at[b], sem_w[t]).wait()

        ga = pltpu.async_copy(
            table_hbm.at[idx_v.at[bl]], slab.at[pl.ds(D_NUM, 2 * L)], sem_a)

        def chunk(k, carry):
            emit_rows(slab, bl, k, L)
            return carry

        lax.fori_loop(0, D_NUM // L, chunk, 0)
        emit_rows(slab, bl, D_NUM // L, D_NUM % L,
                  x_off=D_NUM - L, lane_off=L - D_NUM % L)

        ga.wait()
        pltpu.async_copy(slab.at[pl.ds(0, N_TOK)], out_hbm.at[b], sem_w[t])

    def ring_body(p, carry):
        for t in range(RING):
            do_slab(p, t)
        return carry

    lax.fori_loop(0, BW // RING, ring_body, 0)
    for t in range(RING):
        pltpu.make_async_copy(
            slabs[t].at[pl.ds(0, N_TOK)],
            out_hbm.at[w * BW + (BW - RING + t)], sem_w[t]).wait()


@jax.jit
def _sc_all(xcat3, xT3, weight, cat_table):
    mesh = plsc.VectorSubcoreMesh(
        core_axis_name="c", subcore_axis_name="s", num_cores=NC, num_subcores=NS)
    return pl.kernel(
        _sc_body,
        out_type=jax.ShapeDtypeStruct((B, N_TOK, D_TOKEN), jnp.float32),
        mesh=mesh,
        compiler_params=pltpu.CompilerParams(needs_layout_passes=False),
        scratch_types=[
            pltpu.VMEM((D_NUM, D_TOKEN), jnp.float32),
            pltpu.VMEM((BW, D_NUM), jnp.float32),
            pltpu.VMEM((BW, 2 * L), jnp.int32),
            pltpu.VMEM((SLAB, D_TOKEN), jnp.float32),
            pltpu.VMEM((SLAB, D_TOKEN), jnp.float32),
            pltpu.VMEM((SLAB, D_TOKEN), jnp.float32),
            pltpu.VMEM((SLAB, D_TOKEN), jnp.float32),
            pltpu.SemaphoreType.DMA,
            pltpu.SemaphoreType.DMA,
            pltpu.SemaphoreType.DMA,
            pltpu.SemaphoreType.DMA,
            pltpu.SemaphoreType.DMA,
        ],
    )(xcat3, xT3, weight, cat_table)


def kernel(x_num, x_cat, weight, cat_table):
    xcat3 = jnp.pad(x_cat, ((0, 0), (0, 2 * L - N_CAT))).reshape(NW, BW, 2 * L)
    xT3 = x_num.reshape(NW, BW, D_NUM)
    return _sc_all(xcat3, xT3, weight, cat_table)


# speedup vs baseline: 3.5985x; 3.5985x over previous
import jax
import jax.numpy as jnp
import numpy as np
from jax import lax
from jax.experimental import pallas as pl
from jax.experimental.pallas import tpu as pltpu
from jax.experimental.pallas import tpu_sc as plsc

B = 4096
D_NUM = 100
N_CAT = 26
CAT_SIZE = 1000
D_TOKEN = 128
N_TOK = D_NUM + N_CAT
R = B * N_CAT

NC = 2
NS = 16
NW = NC * NS
PER_W = R // NW
CHUNK = 128
N_CHUNKS = PER_W // CHUNK

_OFFSETS = np.cumsum([0] + [CAT_SIZE] * (N_CAT - 1)).astype(np.int32)
_OFF_PATTERN = np.tile(_OFFSETS, PER_W // N_CAT).reshape(N_CHUNKS, CHUNK)


def _sc_gather_body(xcat_hbm, off_hbm, table_hbm, out_hbm, idx_v, off_v, buf0, buf1, sem0, sem1):
    w = lax.axis_index("c") * NS + lax.axis_index("s")
    base_o = w * PER_W

    pltpu.sync_copy(xcat_hbm.at[w], idx_v)
    pltpu.sync_copy(off_hbm, off_v)

    def add_offsets(r, carry):
        for i in range(CHUNK // 16):
            s = pl.ds(i * 16, 16)
            idx_v[r, s] = idx_v[r, s] + off_v[r, s]
        return carry

    lax.fori_loop(0, N_CHUNKS, add_offsets, 0)

    bufs = (buf0, buf1)
    sems = (sem0, sem1)
    copies = [None, None]
    copies[0] = pltpu.async_copy(table_hbm.at[idx_v.at[0]], bufs[0], sems[0])
    for c in range(N_CHUNKS):
        if c + 1 < N_CHUNKS:
            copies[(c + 1) % 2] = pltpu.async_copy(
                table_hbm.at[idx_v.at[c + 1]], bufs[(c + 1) % 2], sems[(c + 1) % 2])
        copies[c % 2].wait()
        pltpu.sync_copy(bufs[c % 2], out_hbm.at[pl.ds(base_o + c * CHUNK, CHUNK)])


@jax.jit
def _sc_gather(xcat3d, off2d, cat_table):
    mesh = plsc.VectorSubcoreMesh(
        core_axis_name="c", subcore_axis_name="s", num_cores=NC, num_subcores=NS)
    return pl.kernel(
        _sc_gather_body,
        out_type=jax.ShapeDtypeStruct((R, D_TOKEN), jnp.float32),
        mesh=mesh,
        scratch_types=[
            pltpu.VMEM((N_CHUNKS, CHUNK), jnp.int32),
            pltpu.VMEM((N_CHUNKS, CHUNK), jnp.int32),
            pltpu.VMEM((CHUNK, D_TOKEN), jnp.float32),
            pltpu.VMEM((CHUNK, D_TOKEN), jnp.float32),
            pltpu.SemaphoreType.DMA,
            pltpu.SemaphoreType.DMA,
        ],
    )(xcat3d, off2d, cat_table)


BB = 128


def _assemble_body(x_ref, w_ref, cat_ref, out_ref):
    out_ref[:, :D_NUM, :] = x_ref[...][:, :, None] * w_ref[...][None, :, :]
    out_ref[:, D_NUM:, :] = cat_ref[...]


@jax.jit
def _tc_assemble(x_num, weight, cat_tok):
    return pl.pallas_call(
        _assemble_body,
        grid=(B // BB,),
        in_specs=[
            pl.BlockSpec((BB, D_NUM), lambda i: (i, 0)),
            pl.BlockSpec((D_NUM, D_TOKEN), lambda i: (0, 0)),
            pl.BlockSpec((BB, N_CAT, D_TOKEN), lambda i: (i, 0, 0)),
        ],
        out_specs=pl.BlockSpec((BB, N_TOK, D_TOKEN), lambda i: (i, 0, 0)),
        out_shape=jax.ShapeDtypeStruct((B, N_TOK, D_TOKEN), jnp.float32),
    )(x_num, weight, cat_tok)


def kernel(x_num, x_cat, weight, cat_table):
    xcat3d = x_cat.reshape(NW, N_CHUNKS, CHUNK)
    off2d = jnp.asarray(_OFF_PATTERN)
    cat_flat = _sc_gather(xcat3d, off2d, cat_table)
    return _tc_assemble(x_num, weight, cat_flat.reshape(B, N_CAT, D_TOKEN))


# speedup vs baseline: 4.1883x; 1.1639x over previous
import jax
import jax.numpy as jnp
import numpy as np
from jax import lax
from jax.experimental import pallas as pl
from jax.experimental.pallas import tpu as pltpu
from jax.experimental.pallas import tpu_sc as plsc

B = 4096
D_NUM = 100
N_CAT = 26
CAT_SIZE = 1000
D_TOKEN = 128
N_TOK = D_NUM + N_CAT
R = B * N_CAT

NC = 2
NS = 16
NW = NC * NS
PER_W = R // NW
CHUNK = 128
N_CHUNKS = PER_W // CHUNK

_OFFSETS = np.cumsum([0] + [CAT_SIZE] * (N_CAT - 1)).astype(np.int32)
_OFF_PATTERN = np.tile(_OFFSETS, PER_W // N_CAT).reshape(N_CHUNKS, CHUNK)


def _sc_gather_body(xcat_hbm, off_hbm, table_hbm, out_hbm, idx_v, off_v, buf0, buf1, sem0, sem1):
    w = lax.axis_index("c") * NS + lax.axis_index("s")
    base_o = w * PER_W

    pltpu.sync_copy(xcat_hbm.at[w], idx_v)
    pltpu.sync_copy(off_hbm, off_v)

    def add_offsets(r, carry):
        for i in range(CHUNK // 16):
            s = pl.ds(i * 16, 16)
            idx_v[r, s] = idx_v[r, s] + off_v[r, s]
        return carry

    lax.fori_loop(0, N_CHUNKS, add_offsets, 0)

    bufs = (buf0, buf1)
    sems = (sem0, sem1)
    copies = [None, None]
    copies[0] = pltpu.async_copy(table_hbm.at[idx_v.at[0]], bufs[0], sems[0])
    for c in range(N_CHUNKS):
        if c + 1 < N_CHUNKS:
            copies[(c + 1) % 2] = pltpu.async_copy(
                table_hbm.at[idx_v.at[c + 1]], bufs[(c + 1) % 2], sems[(c + 1) % 2])
        copies[c % 2].wait()
        pltpu.sync_copy(bufs[c % 2], out_hbm.at[pl.ds(base_o + c * CHUNK, CHUNK)])


@jax.jit
def _sc_gather(xcat3d, off2d, cat_table):
    mesh = plsc.VectorSubcoreMesh(
        core_axis_name="c", subcore_axis_name="s", num_cores=NC, num_subcores=NS)
    return pl.kernel(
        _sc_gather_body,
        out_type=jax.ShapeDtypeStruct((R, D_TOKEN), jnp.float32),
        mesh=mesh,
        scratch_types=[
            pltpu.VMEM((N_CHUNKS, CHUNK), jnp.int32),
            pltpu.VMEM((N_CHUNKS, CHUNK), jnp.int32),
            pltpu.VMEM((CHUNK, D_TOKEN), jnp.float32),
            pltpu.VMEM((CHUNK, D_TOKEN), jnp.float32),
            pltpu.SemaphoreType.DMA,
            pltpu.SemaphoreType.DMA,
        ],
    )(xcat3d, off2d, cat_table)


BB = 128


def _assemble_body(x_ref, w_ref, cat_ref, out_ref):
    out_ref[:, :D_NUM, :] = x_ref[...][:, :, None] * w_ref[...][None, :, :]
    out_ref[:, D_NUM:, :] = cat_ref[...].reshape(BB, N_CAT, D_TOKEN)


@jax.jit
def _tc_assemble(x_num, weight, cat_tok):
    return pl.pallas_call(
        _assemble_body,
        grid=(B // BB,),
        in_specs=[
            pl.BlockSpec((BB, D_NUM), lambda i: (i, 0)),
            pl.BlockSpec((D_NUM, D_TOKEN), lambda i: (0, 0)),
            pl.BlockSpec((BB * N_CAT, D_TOKEN), lambda i: (i, 0)),
        ],
        out_specs=pl.BlockSpec((BB, N_TOK, D_TOKEN), lambda i: (i, 0, 0)),
        out_shape=jax.ShapeDtypeStruct((B, N_TOK, D_TOKEN), jnp.float32),
    )(x_num, weight, cat_tok)


def kernel(x_num, x_cat, weight, cat_table):
    xcat3d = x_cat.reshape(NW, N_CHUNKS, CHUNK)
    off2d = jnp.asarray(_OFF_PATTERN)
    cat_flat = _sc_gather(xcat3d, off2d, cat_table)
    return _tc_assemble(x_num, weight, cat_flat)


# speedup vs baseline: 4.2044x; 1.0038x over previous
import jax
import jax.numpy as jnp
import numpy as np
from jax import lax
from jax.experimental import pallas as pl
from jax.experimental.pallas import tpu as pltpu
from jax.experimental.pallas import tpu_sc as plsc

B = 4096
D_NUM = 100
N_CAT = 26
CAT_SIZE = 1000
D_TOKEN = 128
N_TOK = D_NUM + N_CAT
R = B * N_CAT

NC = 2
NS = 16
NW = NC * NS
PER_W = R // NW
CHUNK = 128
N_CHUNKS = PER_W // CHUNK

_OFFSETS = np.cumsum([0] + [CAT_SIZE] * (N_CAT - 1)).astype(np.int32)
_OFF_PATTERN = np.tile(_OFFSETS, PER_W // N_CAT).reshape(N_CHUNKS, CHUNK)


def _sc_gather_body(xcat_hbm, off_hbm, table_hbm, out_hbm, idx_v, off_v, buf0, buf1, sem0, sem1):
    w = lax.axis_index("c") * NS + lax.axis_index("s")
    base_o = w * PER_W

    pltpu.sync_copy(xcat_hbm.at[w], idx_v)
    pltpu.sync_copy(off_hbm, off_v)

    def add_offsets(r, carry):
        for i in range(CHUNK // 16):
            s = pl.ds(i * 16, 16)
            idx_v[r, s] = idx_v[r, s] + off_v[r, s]
        return carry

    lax.fori_loop(0, N_CHUNKS, add_offsets, 0)

    bufs = (buf0, buf1)
    sems = (sem0, sem1)
    copies = [None, None]
    copies[0] = pltpu.async_copy(table_hbm.at[idx_v.at[0]], bufs[0], sems[0])
    for c in range(N_CHUNKS):
        if c + 1 < N_CHUNKS:
            copies[(c + 1) % 2] = pltpu.async_copy(
                table_hbm.at[idx_v.at[c + 1]], bufs[(c + 1) % 2], sems[(c + 1) % 2])
        copies[c % 2].wait()
        pltpu.sync_copy(bufs[c % 2], out_hbm.at[pl.ds(base_o + c * CHUNK, CHUNK)])


@jax.jit
def _sc_gather(xcat3d, off2d, cat_table):
    mesh = plsc.VectorSubcoreMesh(
        core_axis_name="c", subcore_axis_name="s", num_cores=NC, num_subcores=NS)
    return pl.kernel(
        _sc_gather_body,
        out_type=jax.ShapeDtypeStruct((R, D_TOKEN), jnp.float32),
        mesh=mesh,
        scratch_types=[
            pltpu.VMEM((N_CHUNKS, CHUNK), jnp.int32),
            pltpu.VMEM((N_CHUNKS, CHUNK), jnp.int32),
            pltpu.VMEM((CHUNK, D_TOKEN), jnp.float32),
            pltpu.VMEM((CHUNK, D_TOKEN), jnp.float32),
            pltpu.SemaphoreType.DMA,
            pltpu.SemaphoreType.DMA,
        ],
    )(xcat3d, off2d, cat_table)


BB = 256


def _assemble_body(x_ref, w_ref, cat_ref, out_ref):
    out_ref[:, :D_NUM, :] = x_ref[...][:, :, None] * w_ref[...][None, :, :]
    out_ref[:, D_NUM:, :] = cat_ref[...].reshape(BB, N_CAT, D_TOKEN)


@jax.jit
def _tc_assemble(x_num, weight, cat_tok):
    return pl.pallas_call(
        _assemble_body,
        grid=(B // BB,),
        in_specs=[
            pl.BlockSpec((BB, D_NUM), lambda i: (i, 0)),
            pl.BlockSpec((D_NUM, D_TOKEN), lambda i: (0, 0)),
            pl.BlockSpec((BB * N_CAT, D_TOKEN), lambda i: (i, 0)),
        ],
        out_specs=pl.BlockSpec((BB, N_TOK, D_TOKEN), lambda i: (i, 0, 0)),
        out_shape=jax.ShapeDtypeStruct((B, N_TOK, D_TOKEN), jnp.float32),
    )(x_num, weight, cat_tok)


def kernel(x_num, x_cat, weight, cat_table):
    xcat3d = x_cat.reshape(NW, N_CHUNKS, CHUNK)
    off2d = jnp.asarray(_OFF_PATTERN)
    cat_flat = _sc_gather(xcat3d, off2d, cat_table)
    return _tc_assemble(x_num, weight, cat_flat)
